# baseline (device time: 13623 ns/iter reference)
import jax
import jax.numpy as jnp
from jax import lax
from jax.experimental import pallas as pl
from jax.experimental.pallas import tpu as pltpu

T = 256
D = 512
V_LOCAL = 4096


def kernel(x, W, labels):
    labels2d = labels.reshape(T, 1)

    def body(x_ref, w_ref, lab_ref, out_ref,
             send_ref, recv_ref, send_sem, recv_sem):
        my_x = lax.axis_index("x")
        my_y = lax.axis_index("y")
        nbr = (1 - my_x, my_y)

        barrier_sem = pltpu.get_barrier_semaphore()
        pl.semaphore_signal(barrier_sem, inc=1, device_id=nbr,
                            device_id_type=pl.DeviceIdType.MESH)
        pl.semaphore_wait(barrier_sem, 1)

        xb = x_ref[:, :].astype(jnp.bfloat16)
        wb = w_ref[:, :].astype(jnp.bfloat16)
        logits = jnp.dot(xb, wb, preferred_element_type=jnp.float32)

        m = jnp.max(logits, axis=1, keepdims=True)
        s = jnp.sum(jnp.exp(logits - m), axis=1, keepdims=True)

        local_idx = lab_ref[:, :] - my_x * V_LOCAL
        col = lax.broadcasted_iota(jnp.int32, (T, V_LOCAL), 1)
        l = jnp.sum(jnp.where(col == local_idx, logits, 0.0),
                    axis=1, keepdims=True)

        send_ref[:, 0:1] = m
        send_ref[:, 1:2] = s
        send_ref[:, 2:3] = l
        send_ref[:, 3:4] = jnp.zeros((T, 1), jnp.float32)

        rdma = pltpu.make_async_remote_copy(
            src_ref=send_ref, dst_ref=recv_ref,
            send_sem=send_sem, recv_sem=recv_sem,
            device_id=nbr, device_id_type=pl.DeviceIdType.MESH,
        )
        rdma.start()
        rdma.wait()

        mo = recv_ref[:, 0:1]
        so = recv_ref[:, 1:2]
        lo = recv_ref[:, 2:3]
        mg = jnp.maximum(m, mo)
        sg = s * jnp.exp(m - mg) + so * jnp.exp(mo - mg)
        out_ref[:, :] = mg + jnp.log(sg) - (l + lo)

    out = pl.pallas_call(
        body,
        out_shape=jax.ShapeDtypeStruct((T, 1), jnp.float32),
        in_specs=[
            pl.BlockSpec(memory_space=pltpu.VMEM),
            pl.BlockSpec(memory_space=pltpu.VMEM),
            pl.BlockSpec(memory_space=pltpu.VMEM),
        ],
        out_specs=pl.BlockSpec(memory_space=pltpu.VMEM),
        scratch_shapes=[
            pltpu.VMEM((T, 4), jnp.float32),
            pltpu.VMEM((T, 4), jnp.float32),
            pltpu.SemaphoreType.DMA,
            pltpu.SemaphoreType.DMA,
        ],
        compiler_params=pltpu.CompilerParams(collective_id=0),
    )(x, W, labels2d)
    return out.reshape(T)
